# baseline (speedup 1.0000x reference)
import functools

import jax
import jax.numpy as jnp
from jax import lax
from jax.experimental import pallas as pl
from jax.experimental.pallas import tpu as pltpu

BASE_GSD = 10
EMBED = 256
DIMQ = EMBED // 4
B, H, W, T, BS = 4, 16, 16, 12, 3


def _month_table():
    months = jnp.arange(12, dtype=jnp.float32)
    angles = 2.0 * jnp.pi * months / 12.0
    half = DIMQ // 2
    freq = jnp.arange(1, half + 1, dtype=jnp.float32)
    arg = angles[:, None] * freq[None, :]
    return jnp.concatenate([jnp.sin(arg), jnp.cos(arg)], axis=-1)


def _pos_embed():
    half = DIMQ // 2
    omega = 1.0 / (10000.0 ** (jnp.arange(half, dtype=jnp.float32) / half))
    out = jnp.arange(T, dtype=jnp.float32)[:, None] * omega
    return jnp.concatenate([jnp.sin(out), jnp.cos(out)], axis=-1)


def _embed_add_kernel(tok_ref, months_ref, mtab_ref, pe_ref, ch_ref, gsd_ref,
                      out_ref):
    h = pl.program_id(1)
    tok = tok_ref[0, 0]
    months = months_ref[0]
    mtab = mtab_ref[...]
    pe = pe_ref[...]
    ch = ch_ref[...]
    gsd = gsd_ref[0, 0]

    oh = (months == lax.broadcasted_iota(jnp.int32, (T, 12), 1))
    me = jnp.dot(oh.astype(jnp.float32), mtab,
                 preferred_element_type=jnp.float32)

    half2 = DIMQ // 4
    kf = lax.broadcasted_iota(jnp.int32, (1, half2), 1).astype(jnp.float32)
    omega = jnp.exp(kf * (-jnp.log(10000.0) / half2))
    posf = lax.broadcasted_iota(jnp.int32, (H, 1), 0).astype(jnp.float32) * gsd
    arg = posf * omega
    E = jnp.concatenate([jnp.sin(arg), jnp.cos(arg)], axis=-1)
    argh = h.astype(jnp.float32) * gsd * omega
    Eh = jnp.concatenate([jnp.sin(argh), jnp.cos(argh)], axis=-1)
    S_h = jnp.concatenate(
        [jnp.broadcast_to(Eh, (W, DIMQ // 2)), E], axis=-1)

    core = jnp.concatenate([pe, me], axis=-1)
    zq = jnp.zeros((T, DIMQ), jnp.float32)
    ts = jnp.concatenate(
        [jnp.concatenate(
            [jnp.broadcast_to(ch[s:s + 1], (T, DIMQ)), core, zq], axis=-1)
         for s in range(BS)], axis=-1)
    z3 = jnp.zeros((W, 3 * DIMQ), jnp.float32)
    sp = jnp.concatenate([z3, S_h] * BS, axis=-1)

    out_ref[0, 0] = tok + ts[None, :, :] + sp[:, None, :]


def kernel(modality_tokens, timestamps, channel_embed, patch_size, input_res):
    b, h, w, t, b_s, d = modality_tokens.shape
    tok = modality_tokens.reshape(b, h, w, t, b_s * d)
    months = timestamps[:, :, 1].reshape(b, t, 1)
    gsd = (jnp.asarray(input_res).astype(jnp.float32)
           * jnp.asarray(patch_size).astype(jnp.float32)
           / float(BASE_GSD)).reshape(1, 1)
    mtab = _month_table()
    pe = _pos_embed()

    out = pl.pallas_call(
        _embed_add_kernel,
        grid=(b, h),
        in_specs=[
            pl.BlockSpec((1, 1, w, t, b_s * d), lambda i, j: (i, j, 0, 0, 0)),
            pl.BlockSpec((1, t, 1), lambda i, j: (i, 0, 0)),
            pl.BlockSpec((12, DIMQ), lambda i, j: (0, 0)),
            pl.BlockSpec((t, DIMQ), lambda i, j: (0, 0)),
            pl.BlockSpec((b_s, DIMQ), lambda i, j: (0, 0)),
            pl.BlockSpec((1, 1), lambda i, j: (0, 0)),
        ],
        out_specs=pl.BlockSpec((1, 1, w, t, b_s * d),
                               lambda i, j: (i, j, 0, 0, 0)),
        out_shape=jax.ShapeDtypeStruct((b, h, w, t, b_s * d), jnp.float32),
        compiler_params=pltpu.CompilerParams(
            dimension_semantics=("parallel", "parallel")),
    )(tok, months, mtab, pe, channel_embed, gsd)
    return out.reshape(b, h, w, t, b_s, d)


# speedup vs baseline: 1.0688x; 1.0688x over previous
import functools

import jax
import jax.numpy as jnp
from jax import lax
from jax.experimental import pallas as pl
from jax.experimental.pallas import tpu as pltpu

BASE_GSD = 10
EMBED = 256
DIMQ = EMBED // 4
B, H, W, T, BS = 4, 16, 16, 12, 3


def _month_table():
    months = jnp.arange(12, dtype=jnp.float32)
    angles = 2.0 * jnp.pi * months / 12.0
    half = DIMQ // 2
    freq = jnp.arange(1, half + 1, dtype=jnp.float32)
    arg = angles[:, None] * freq[None, :]
    return jnp.concatenate([jnp.sin(arg), jnp.cos(arg)], axis=-1)


def _pos_embed():
    half = DIMQ // 2
    omega = 1.0 / (10000.0 ** (jnp.arange(half, dtype=jnp.float32) / half))
    out = jnp.arange(T, dtype=jnp.float32)[:, None] * omega
    return jnp.concatenate([jnp.sin(out), jnp.cos(out)], axis=-1)


def _embed_add_kernel(tok_ref, months_ref, mtab_ref, pe_ref, ch_ref, gsd_ref,
                      out_ref):
    h = pl.program_id(1)
    months = months_ref[0]
    mtab = mtab_ref[...]
    pe = pe_ref[...]
    ch = ch_ref[...]
    gsd = gsd_ref[0, 0]

    oh = (months == lax.broadcasted_iota(jnp.int32, (T, 12), 1))
    me = jnp.dot(oh.astype(jnp.float32), mtab,
                 preferred_element_type=jnp.float32)

    half2 = DIMQ // 4
    kf = lax.broadcasted_iota(jnp.int32, (1, half2), 1).astype(jnp.float32)
    omega = jnp.exp(kf * (-jnp.log(10000.0) / half2))
    posf = lax.broadcasted_iota(jnp.int32, (W, 1), 0).astype(jnp.float32) * gsd
    arg = posf * omega
    E = jnp.concatenate([jnp.sin(arg), jnp.cos(arg)], axis=-1)
    argh = h.astype(jnp.float32) * gsd * omega
    Eh = jnp.concatenate([jnp.sin(argh), jnp.cos(argh)], axis=-1)
    S_h = jnp.concatenate(
        [jnp.broadcast_to(Eh, (W, DIMQ // 2)), E], axis=-1)

    z64 = jnp.zeros((1, DIMQ), jnp.float32)
    pieces = []
    for t in range(T):
        for s in range(BS):
            pieces += [ch[s:s + 1], pe[t:t + 1], me[t:t + 1], z64]
    ts_flat = jnp.concatenate(pieces, axis=-1)
    z3 = jnp.zeros((W, 3 * DIMQ), jnp.float32)
    sp = jnp.concatenate([z3, S_h] * (T * BS), axis=-1)

    out_ref[0, 0] = tok_ref[0, 0] + ts_flat + sp


def kernel(modality_tokens, timestamps, channel_embed, patch_size, input_res):
    b, h, w, t, b_s, d = modality_tokens.shape
    n = t * b_s * d
    tok = modality_tokens.reshape(b, h, w, n)
    months = timestamps[:, :, 1].reshape(b, t, 1)
    gsd = (jnp.asarray(input_res).astype(jnp.float32)
           * jnp.asarray(patch_size).astype(jnp.float32)
           / float(BASE_GSD)).reshape(1, 1)
    mtab = _month_table()
    pe = _pos_embed()

    out = pl.pallas_call(
        _embed_add_kernel,
        grid=(b, h),
        in_specs=[
            pl.BlockSpec((1, 1, w, n), lambda i, j: (i, j, 0, 0)),
            pl.BlockSpec((1, t, 1), lambda i, j: (i, 0, 0)),
            pl.BlockSpec((12, DIMQ), lambda i, j: (0, 0)),
            pl.BlockSpec((t, DIMQ), lambda i, j: (0, 0)),
            pl.BlockSpec((b_s, DIMQ), lambda i, j: (0, 0)),
            pl.BlockSpec((1, 1), lambda i, j: (0, 0)),
        ],
        out_specs=pl.BlockSpec((1, 1, w, n), lambda i, j: (i, j, 0, 0)),
        out_shape=jax.ShapeDtypeStruct((b, h, w, n), jnp.float32),
        compiler_params=pltpu.CompilerParams(
            dimension_semantics=("parallel", "parallel")),
    )(tok, months, mtab, pe, channel_embed, gsd)
    return out.reshape(b, h, w, t, b_s, d)


# speedup vs baseline: 1.7387x; 1.6268x over previous
import functools

import jax
import jax.numpy as jnp
from jax import lax
from jax.experimental import pallas as pl
from jax.experimental.pallas import tpu as pltpu

BASE_GSD = 10
EMBED = 256
DIMQ = EMBED // 4
B, H, W, T, BS = 4, 16, 16, 12, 3


def _month_table():
    months = jnp.arange(12, dtype=jnp.float32)
    angles = 2.0 * jnp.pi * months / 12.0
    half = DIMQ // 2
    freq = jnp.arange(1, half + 1, dtype=jnp.float32)
    arg = angles[:, None] * freq[None, :]
    return jnp.concatenate([jnp.sin(arg), jnp.cos(arg)], axis=-1)


def _pos_embed():
    half = DIMQ // 2
    omega = 1.0 / (10000.0 ** (jnp.arange(half, dtype=jnp.float32) / half))
    out = jnp.arange(T, dtype=jnp.float32)[:, None] * omega
    return jnp.concatenate([jnp.sin(out), jnp.cos(out)], axis=-1)


def _embed_add_kernel(tok_ref, months_ref, mtab_ref, pe_ref, ch_ref, gsd_ref,
                      out_ref):
    h = pl.program_id(1)
    months = months_ref[0]
    mtab = mtab_ref[...]
    pe = pe_ref[...]
    ch = ch_ref[...]
    gsd = gsd_ref[0, 0]

    oh = (months == lax.broadcasted_iota(jnp.int32, (T, 12), 1))
    me = jnp.dot(oh.astype(jnp.float32), mtab,
                 preferred_element_type=jnp.float32)

    half2 = DIMQ // 4
    kf = lax.broadcasted_iota(jnp.int32, (1, half2), 1).astype(jnp.float32)
    omega = jnp.exp(kf * (-jnp.log(10000.0) / half2))
    posf = lax.broadcasted_iota(jnp.int32, (W, 1), 0).astype(jnp.float32) * gsd
    arg = posf * omega
    E = jnp.concatenate([jnp.sin(arg), jnp.cos(arg)], axis=-1)
    argh = h.astype(jnp.float32) * gsd * omega
    Eh = jnp.concatenate([jnp.sin(argh), jnp.cos(argh)], axis=-1)
    S_h = jnp.concatenate(
        [jnp.broadcast_to(Eh, (W, DIMQ // 2)), E], axis=-1)

    add_ts = jnp.concatenate([
        jnp.broadcast_to(ch[None, :, :], (T, BS, DIMQ)),
        jnp.broadcast_to(pe[:, None, :], (T, BS, DIMQ)),
        jnp.broadcast_to(me[:, None, :], (T, BS, DIMQ)),
        jnp.zeros((T, BS, DIMQ), jnp.float32),
    ], axis=-1)
    sp_w = jnp.concatenate(
        [jnp.zeros((W, 3 * DIMQ), jnp.float32), S_h], axis=-1)

    out_ref[0, 0] = (tok_ref[0, 0] + add_ts[None, :, :, :]
                     + sp_w[:, None, None, :])


def kernel(modality_tokens, timestamps, channel_embed, patch_size, input_res):
    b, h, w, t, b_s, d = modality_tokens.shape
    months = timestamps[:, :, 1].reshape(b, t, 1)
    gsd = (jnp.asarray(input_res).astype(jnp.float32)
           * jnp.asarray(patch_size).astype(jnp.float32)
           / float(BASE_GSD)).reshape(1, 1)
    mtab = _month_table()
    pe = _pos_embed()

    return pl.pallas_call(
        _embed_add_kernel,
        grid=(b, h),
        in_specs=[
            pl.BlockSpec((1, 1, w, t, b_s, d),
                         lambda i, j: (i, j, 0, 0, 0, 0)),
            pl.BlockSpec((1, t, 1), lambda i, j: (i, 0, 0)),
            pl.BlockSpec((12, DIMQ), lambda i, j: (0, 0)),
            pl.BlockSpec((t, DIMQ), lambda i, j: (0, 0)),
            pl.BlockSpec((b_s, DIMQ), lambda i, j: (0, 0)),
            pl.BlockSpec((1, 1), lambda i, j: (0, 0)),
        ],
        out_specs=pl.BlockSpec((1, 1, w, t, b_s, d),
                               lambda i, j: (i, j, 0, 0, 0, 0)),
        out_shape=jax.ShapeDtypeStruct((b, h, w, t, b_s, d), jnp.float32),
        compiler_params=pltpu.CompilerParams(
            dimension_semantics=("parallel", "parallel")),
    )(modality_tokens, months, mtab, pe, channel_embed, gsd)


# speedup vs baseline: 2.1315x; 1.2259x over previous
import functools

import jax
import jax.numpy as jnp
from jax import lax
from jax.experimental import pallas as pl
from jax.experimental.pallas import tpu as pltpu

BASE_GSD = 10
EMBED = 256
DIMQ = EMBED // 4
B, H, W, T, BS = 4, 16, 16, 12, 3


def _month_table():
    months = jnp.arange(12, dtype=jnp.float32)
    angles = 2.0 * jnp.pi * months / 12.0
    half = DIMQ // 2
    freq = jnp.arange(1, half + 1, dtype=jnp.float32)
    arg = angles[:, None] * freq[None, :]
    return jnp.concatenate([jnp.sin(arg), jnp.cos(arg)], axis=-1)


def _pos_embed():
    half = DIMQ // 2
    omega = 1.0 / (10000.0 ** (jnp.arange(half, dtype=jnp.float32) / half))
    out = jnp.arange(T, dtype=jnp.float32)[:, None] * omega
    return jnp.concatenate([jnp.sin(out), jnp.cos(out)], axis=-1)


HH = 4


def _embed_add_kernel(tok_ref, months_ref, mtab_ref, pe_ref, ch_ref, gsd_ref,
                      out_ref):
    j = pl.program_id(1)
    months = months_ref[0]
    mtab = mtab_ref[...]
    pe = pe_ref[...]
    ch = ch_ref[...]
    gsd = gsd_ref[0, 0]

    oh = (months == lax.broadcasted_iota(jnp.int32, (T, 12), 1))
    me = jnp.dot(oh.astype(jnp.float32), mtab,
                 preferred_element_type=jnp.float32)

    half2 = DIMQ // 4
    kf = lax.broadcasted_iota(jnp.int32, (1, half2), 1).astype(jnp.float32)
    omega = jnp.exp(kf * (-jnp.log(10000.0) / half2))
    posf = lax.broadcasted_iota(jnp.int32, (W, 1), 0).astype(jnp.float32) * gsd
    arg = posf * omega
    E = jnp.concatenate([jnp.sin(arg), jnp.cos(arg)], axis=-1)
    hrow = (j * HH + lax.broadcasted_iota(jnp.int32, (HH, 1), 0))
    argh = hrow.astype(jnp.float32) * gsd * omega
    Eh = jnp.concatenate([jnp.sin(argh), jnp.cos(argh)], axis=-1)
    S4 = jnp.concatenate([
        jnp.broadcast_to(Eh[:, None, :], (HH, W, DIMQ // 2)),
        jnp.broadcast_to(E[None, :, :], (HH, W, DIMQ // 2)),
    ], axis=-1)

    add_ts = jnp.concatenate([
        jnp.broadcast_to(ch[None, :, :], (T, BS, DIMQ)),
        jnp.broadcast_to(pe[:, None, :], (T, BS, DIMQ)),
        jnp.broadcast_to(me[:, None, :], (T, BS, DIMQ)),
        jnp.zeros((T, BS, DIMQ), jnp.float32),
    ], axis=-1)
    sp_hw = jnp.concatenate(
        [jnp.zeros((HH, W, 3 * DIMQ), jnp.float32), S4], axis=-1)

    out_ref[0] = (tok_ref[0] + add_ts[None, None, :, :, :]
                  + sp_hw[:, :, None, None, :])


def kernel(modality_tokens, timestamps, channel_embed, patch_size, input_res):
    b, h, w, t, b_s, d = modality_tokens.shape
    months = timestamps[:, :, 1].reshape(b, t, 1)
    gsd = (jnp.asarray(input_res).astype(jnp.float32)
           * jnp.asarray(patch_size).astype(jnp.float32)
           / float(BASE_GSD)).reshape(1, 1)
    mtab = _month_table()
    pe = _pos_embed()

    return pl.pallas_call(
        _embed_add_kernel,
        grid=(b, h // HH),
        in_specs=[
            pl.BlockSpec((1, HH, w, t, b_s, d),
                         lambda i, j: (i, j, 0, 0, 0, 0)),
            pl.BlockSpec((1, t, 1), lambda i, j: (i, 0, 0)),
            pl.BlockSpec((12, DIMQ), lambda i, j: (0, 0)),
            pl.BlockSpec((t, DIMQ), lambda i, j: (0, 0)),
            pl.BlockSpec((b_s, DIMQ), lambda i, j: (0, 0)),
            pl.BlockSpec((1, 1), lambda i, j: (0, 0)),
        ],
        out_specs=pl.BlockSpec((1, HH, w, t, b_s, d),
                               lambda i, j: (i, j, 0, 0, 0, 0)),
        out_shape=jax.ShapeDtypeStruct((b, h, w, t, b_s, d), jnp.float32),
        compiler_params=pltpu.CompilerParams(
            dimension_semantics=("parallel", "parallel")),
    )(modality_tokens, months, mtab, pe, channel_embed, gsd)


# speedup vs baseline: 2.1406x; 1.0043x over previous
import functools

import jax
import jax.numpy as jnp
from jax import lax
from jax.experimental import pallas as pl
from jax.experimental.pallas import tpu as pltpu

BASE_GSD = 10
EMBED = 256
DIMQ = EMBED // 4
B, H, W, T, BS = 4, 16, 16, 12, 3
NCHUNK = B * H
NB = 4


def _month_table():
    months = jnp.arange(12, dtype=jnp.float32)
    angles = 2.0 * jnp.pi * months / 12.0
    half = DIMQ // 2
    freq = jnp.arange(1, half + 1, dtype=jnp.float32)
    arg = angles[:, None] * freq[None, :]
    return jnp.concatenate([jnp.sin(arg), jnp.cos(arg)], axis=-1)


def _pos_embed():
    half = DIMQ // 2
    omega = 1.0 / (10000.0 ** (jnp.arange(half, dtype=jnp.float32) / half))
    out = jnp.arange(T, dtype=jnp.float32)[:, None] * omega
    return jnp.concatenate([jnp.sin(out), jnp.cos(out)], axis=-1)


def _embed_add_kernel(tok_ref, months_ref, mtab_ref, pe_ref, ch_ref, gsd_ref,
                      out_ref, in_buf, out_buf, add_ts_ref, sp_ref,
                      in_sem, out_sem):
    months = months_ref[...]
    mtab = mtab_ref[...]
    pe = pe_ref[...]
    ch = ch_ref[...]
    gsd = gsd_ref[0, 0]

    for b_i in range(B):
        oh = (months[b_i] == lax.broadcasted_iota(jnp.int32, (T, 12), 1))
        me = jnp.dot(oh.astype(jnp.float32), mtab,
                     preferred_element_type=jnp.float32)
        add_ts_ref[b_i] = jnp.concatenate([
            jnp.broadcast_to(ch[None, :, :], (T, BS, DIMQ)),
            jnp.broadcast_to(pe[:, None, :], (T, BS, DIMQ)),
            jnp.broadcast_to(me[:, None, :], (T, BS, DIMQ)),
            jnp.zeros((T, BS, DIMQ), jnp.float32),
        ], axis=-1)

    half2 = DIMQ // 4
    kf = lax.broadcasted_iota(jnp.int32, (1, half2), 1).astype(jnp.float32)
    omega = jnp.exp(kf * (-jnp.log(10000.0) / half2))
    posf = lax.broadcasted_iota(jnp.int32, (H, 1), 0).astype(jnp.float32) * gsd
    arg = posf * omega
    E = jnp.concatenate([jnp.sin(arg), jnp.cos(arg)], axis=-1)
    S = jnp.concatenate([
        jnp.broadcast_to(E[:, None, :], (H, W, DIMQ // 2)),
        jnp.broadcast_to(E[None, :, :], (H, W, DIMQ // 2)),
    ], axis=-1)
    sp_ref[...] = jnp.concatenate(
        [jnp.zeros((H, W, 3 * DIMQ), jnp.float32), S], axis=-1)

    def in_copy(k, slot):
        return pltpu.make_async_copy(
            tok_ref.at[k // H, lax.rem(k, H)], in_buf.at[slot],
            in_sem.at[slot])

    def out_copy(k, slot):
        return pltpu.make_async_copy(
            out_buf.at[slot], out_ref.at[k // H, lax.rem(k, H)],
            out_sem.at[slot])

    for j in range(NB):
        in_copy(j, j).start()

    def body(k, carry):
        slot = lax.rem(k, NB)
        in_copy(k, slot).wait()

        @pl.when(k >= NB)
        def _():
            out_copy(k - NB, slot).wait()

        b_i = k // H
        h_i = lax.rem(k, H)
        out_buf[slot] = (in_buf[slot] + add_ts_ref[b_i][None]
                         + sp_ref[h_i][:, None, None, :])
        out_copy(k, slot).start()

        @pl.when(k + NB < NCHUNK)
        def _():
            in_copy(k + NB, slot).start()

        return carry

    lax.fori_loop(0, NCHUNK, body, 0)

    def drain(j, carry):
        k = NCHUNK - NB + j
        out_copy(k, lax.rem(k, NB)).wait()
        return carry

    lax.fori_loop(0, NB, drain, 0)


def kernel(modality_tokens, timestamps, channel_embed, patch_size, input_res):
    b, h, w, t, b_s, d = modality_tokens.shape
    months = timestamps[:, :, 1].reshape(b, t, 1)
    gsd = (jnp.asarray(input_res).astype(jnp.float32)
           * jnp.asarray(patch_size).astype(jnp.float32)
           / float(BASE_GSD)).reshape(1, 1)
    mtab = _month_table()
    pe = _pos_embed()

    return pl.pallas_call(
        _embed_add_kernel,
        in_specs=[
            pl.BlockSpec(memory_space=pl.ANY),
            pl.BlockSpec(memory_space=pltpu.VMEM),
            pl.BlockSpec(memory_space=pltpu.VMEM),
            pl.BlockSpec(memory_space=pltpu.VMEM),
            pl.BlockSpec(memory_space=pltpu.VMEM),
            pl.BlockSpec(memory_space=pltpu.VMEM),
        ],
        out_specs=pl.BlockSpec(memory_space=pl.ANY),
        out_shape=jax.ShapeDtypeStruct((b, h, w, t, b_s, d), jnp.float32),
        scratch_shapes=[
            pltpu.VMEM((NB, W, T, BS, EMBED), jnp.float32),
            pltpu.VMEM((NB, W, T, BS, EMBED), jnp.float32),
            pltpu.VMEM((B, T, BS, EMBED), jnp.float32),
            pltpu.VMEM((H, W, EMBED), jnp.float32),
            pltpu.SemaphoreType.DMA((NB,)),
            pltpu.SemaphoreType.DMA((NB,)),
        ],
    )(modality_tokens, months, mtab, pe, channel_embed, gsd)


# speedup vs baseline: 2.4654x; 1.1517x over previous
import functools

import jax
import jax.numpy as jnp
from jax import lax
from jax.experimental import pallas as pl
from jax.experimental.pallas import tpu as pltpu

BASE_GSD = 10
EMBED = 256
DIMQ = EMBED // 4
B, H, W, T, BS = 4, 16, 16, 12, 3
HH = 4


def _month_table():
    months = jnp.arange(12, dtype=jnp.float32)
    angles = 2.0 * jnp.pi * months / 12.0
    half = DIMQ // 2
    freq = jnp.arange(1, half + 1, dtype=jnp.float32)
    arg = angles[:, None] * freq[None, :]
    return jnp.concatenate([jnp.sin(arg), jnp.cos(arg)], axis=-1)


def _pos_embed():
    half = DIMQ // 2
    omega = 1.0 / (10000.0 ** (jnp.arange(half, dtype=jnp.float32) / half))
    out = jnp.arange(T, dtype=jnp.float32)[:, None] * omega
    return jnp.concatenate([jnp.sin(out), jnp.cos(out)], axis=-1)


def _embed_add_kernel(tok_ref, months_ref, mtab_ref, pe_ref, ch_ref, gsd_ref,
                      out_ref):
    j = pl.program_id(1)
    months = months_ref[0]
    mtab = mtab_ref[...]
    pe = pe_ref[...]
    ch = ch_ref[...]
    gsd = gsd_ref[0, 0]

    oh = (months == lax.broadcasted_iota(jnp.int32, (T, 12), 1))
    me = jnp.dot(oh.astype(jnp.float32), mtab,
                 preferred_element_type=jnp.float32)

    add36 = jnp.concatenate([
        jnp.tile(ch, (T, 1)),
        jnp.repeat(pe, BS, axis=0),
        jnp.repeat(me, BS, axis=0),
        jnp.zeros((T * BS, DIMQ), jnp.float32),
    ], axis=-1)

    half2 = DIMQ // 4
    kf = lax.broadcasted_iota(jnp.int32, (1, half2), 1).astype(jnp.float32)
    omega = jnp.exp(kf * (-jnp.log(10000.0) / half2))
    posf = lax.broadcasted_iota(jnp.int32, (W, 1), 0).astype(jnp.float32) * gsd
    arg = posf * omega
    E = jnp.concatenate([jnp.sin(arg), jnp.cos(arg)], axis=-1)
    hrow = (j * HH + lax.broadcasted_iota(jnp.int32, (HH, 1), 0))
    argh = hrow.astype(jnp.float32) * gsd * omega
    Eh = jnp.concatenate([jnp.sin(argh), jnp.cos(argh)], axis=-1)
    S4 = jnp.concatenate([
        jnp.broadcast_to(Eh[:, None, :], (HH, W, DIMQ // 2)),
        jnp.broadcast_to(E[None, :, :], (HH, W, DIMQ // 2)),
    ], axis=-1)
    sp_hw = jnp.concatenate(
        [jnp.zeros((HH, W, 3 * DIMQ), jnp.float32), S4], axis=-1)

    out_ref[0] = (tok_ref[0] + add36[None, None, :, :]
                  + sp_hw[:, :, None, :])


def kernel(modality_tokens, timestamps, channel_embed, patch_size, input_res):
    b, h, w, t, b_s, d = modality_tokens.shape
    tok = modality_tokens.reshape(b, h, w, t * b_s, d)
    months = timestamps[:, :, 1].reshape(b, t, 1)
    gsd = (jnp.asarray(input_res).astype(jnp.float32)
           * jnp.asarray(patch_size).astype(jnp.float32)
           / float(BASE_GSD)).reshape(1, 1)
    mtab = _month_table()
    pe = _pos_embed()

    out = pl.pallas_call(
        _embed_add_kernel,
        grid=(b, h // HH),
        in_specs=[
            pl.BlockSpec((1, HH, w, t * b_s, d),
                         lambda i, j: (i, j, 0, 0, 0)),
            pl.BlockSpec((1, t, 1), lambda i, j: (i, 0, 0)),
            pl.BlockSpec((12, DIMQ), lambda i, j: (0, 0)),
            pl.BlockSpec((t, DIMQ), lambda i, j: (0, 0)),
            pl.BlockSpec((b_s, DIMQ), lambda i, j: (0, 0)),
            pl.BlockSpec((1, 1), lambda i, j: (0, 0)),
        ],
        out_specs=pl.BlockSpec((1, HH, w, t * b_s, d),
                               lambda i, j: (i, j, 0, 0, 0)),
        out_shape=jax.ShapeDtypeStruct((b, h, w, t * b_s, d), jnp.float32),
        compiler_params=pltpu.CompilerParams(
            dimension_semantics=("parallel", "parallel")),
    )(tok, months, mtab, pe, channel_embed, gsd)
    return out.reshape(b, h, w, t, b_s, d)


# speedup vs baseline: 8.4997x; 3.4476x over previous
import functools

import jax
import jax.numpy as jnp
from jax import lax
from jax.experimental import pallas as pl
from jax.experimental.pallas import tpu as pltpu

BASE_GSD = 10
EMBED = 256
DIMQ = EMBED // 4
B, H, W, T, BS = 4, 16, 16, 12, 3
HH = 4


def _month_table():
    months = jnp.arange(12, dtype=jnp.float32)
    angles = 2.0 * jnp.pi * months / 12.0
    half = DIMQ // 2
    freq = jnp.arange(1, half + 1, dtype=jnp.float32)
    arg = angles[:, None] * freq[None, :]
    return jnp.concatenate([jnp.sin(arg), jnp.cos(arg)], axis=-1)


def _pos_embed():
    half = DIMQ // 2
    omega = 1.0 / (10000.0 ** (jnp.arange(half, dtype=jnp.float32) / half))
    out = jnp.arange(T, dtype=jnp.float32)[:, None] * omega
    return jnp.concatenate([jnp.sin(out), jnp.cos(out)], axis=-1)


def _embed_add_kernel(tok_ref, months_ref, mtab_ref, pe_ref, ch_ref, gsd_ref,
                      out_ref):
    j = pl.program_id(1)
    months = months_ref[0]
    mtab = mtab_ref[...]
    pe = pe_ref[...]
    ch = ch_ref[...]
    gsd = gsd_ref[0, 0]

    oh = (months == lax.broadcasted_iota(jnp.int32, (T, 12), 1))
    me = jnp.dot(oh.astype(jnp.float32), mtab,
                 preferred_element_type=jnp.float32)

    add_ts = jnp.concatenate([
        jnp.broadcast_to(ch[None, :, :], (T, BS, DIMQ)),
        jnp.broadcast_to(pe[:, None, :], (T, BS, DIMQ)),
        jnp.broadcast_to(me[:, None, :], (T, BS, DIMQ)),
        jnp.zeros((T, BS, DIMQ), jnp.float32),
    ], axis=-1)

    half2 = DIMQ // 4
    kf = lax.broadcasted_iota(jnp.int32, (1, half2), 1).astype(jnp.float32)
    omega = jnp.exp(kf * (-jnp.log(10000.0) / half2))
    posf = lax.broadcasted_iota(jnp.int32, (W, 1), 0).astype(jnp.float32) * gsd
    arg = posf * omega
    E = jnp.concatenate([jnp.sin(arg), jnp.cos(arg)], axis=-1)
    hrow = (j * HH + lax.broadcasted_iota(jnp.int32, (HH, 1), 0))
    argh = hrow.astype(jnp.float32) * gsd * omega
    Eh = jnp.concatenate([jnp.sin(argh), jnp.cos(argh)], axis=-1)
    S4 = jnp.concatenate([
        jnp.broadcast_to(Eh[:, None, :], (HH, W, DIMQ // 2)),
        jnp.broadcast_to(E[None, :, :], (HH, W, DIMQ // 2)),
    ], axis=-1)
    sp_hw = jnp.concatenate(
        [jnp.zeros((HH, W, 3 * DIMQ), jnp.float32), S4], axis=-1)

    out_ref[0] = (tok_ref[0] + add_ts[None, :, :, None, :]
                  + sp_hw[:, None, None, :, :])


def kernel(modality_tokens, timestamps, channel_embed, patch_size, input_res):
    b, h, w, t, b_s, d = modality_tokens.shape
    tok = jnp.transpose(modality_tokens, (0, 1, 3, 4, 2, 5))
    months = timestamps[:, :, 1].reshape(b, t, 1)
    gsd = (jnp.asarray(input_res).astype(jnp.float32)
           * jnp.asarray(patch_size).astype(jnp.float32)
           / float(BASE_GSD)).reshape(1, 1)
    mtab = _month_table()
    pe = _pos_embed()

    out = pl.pallas_call(
        _embed_add_kernel,
        grid=(b, h // HH),
        in_specs=[
            pl.BlockSpec((1, HH, t, b_s, w, d),
                         lambda i, j: (i, j, 0, 0, 0, 0)),
            pl.BlockSpec((1, t, 1), lambda i, j: (i, 0, 0)),
            pl.BlockSpec((12, DIMQ), lambda i, j: (0, 0)),
            pl.BlockSpec((t, DIMQ), lambda i, j: (0, 0)),
            pl.BlockSpec((b_s, DIMQ), lambda i, j: (0, 0)),
            pl.BlockSpec((1, 1), lambda i, j: (0, 0)),
        ],
        out_specs=pl.BlockSpec((1, HH, t, b_s, w, d),
                               lambda i, j: (i, j, 0, 0, 0, 0)),
        out_shape=jax.ShapeDtypeStruct((b, h, t, b_s, w, d), jnp.float32),
        compiler_params=pltpu.CompilerParams(
            dimension_semantics=("parallel", "parallel")),
    )(tok, months, mtab, pe, channel_embed, gsd)
    return jnp.transpose(out, (0, 1, 4, 2, 3, 5))
